# baseline (speedup 1.0000x reference)
import functools

import jax
import jax.numpy as jnp
from jax import lax
from jax.experimental import pallas as pl
from jax.experimental.pallas import tpu as pltpu
from jax.experimental.pallas import tpu_sc as plsc

B = 16384
D = 64
NC = 2
NS = 16
NW = NC * NS
BPW = B // NW
WAVE = 128
NWAVE = BPW // WAVE
NBROW = 7813


def _body(u_idx_hbm, i_idx_hbm, ut2_hbm, ubp_hbm, it2_hbm, ibp_hbm,
          out_hbm,
          iv_u, iv_i, r2u, r2i, r3u, r3i,
          gbu, gbi, gbub, gbib, out_v, sem):
    wid = lax.axis_index("s") * NC + lax.axis_index("c")
    base = wid * BPW

    pltpu.sync_copy(u_idx_hbm.at[pl.ds(base, BPW)], iv_u)
    pltpu.sync_copy(i_idx_hbm.at[pl.ds(base, BPW)], iv_i)

    for k in range(BPW // 16):
        w, off = k // (WAVE // 16), (k % (WAVE // 16)) * 16
        sl = pl.ds(k * 16, 16)
        dsl = pl.ds(off, 16)
        u16 = iv_u[sl]
        i16 = iv_i[sl]
        r2u[w, dsl] = lax.shift_right_logical(u16, 1)
        r2i[w, dsl] = lax.shift_right_logical(i16, 1)
        r3u[w, dsl] = lax.shift_right_logical(u16, 7)
        r3i[w, dsl] = lax.shift_right_logical(i16, 7)

    lanes = lax.iota(jnp.int32, 16)

    for w in range(NWAVE):
        cps = (
            pltpu.make_async_copy(ut2_hbm.at[r2u.at[w]], gbu, sem),
            pltpu.make_async_copy(it2_hbm.at[r2i.at[w]], gbi, sem),
            pltpu.make_async_copy(ubp_hbm.at[r3u.at[w]], gbub, sem),
            pltpu.make_async_copy(ibp_hbm.at[r3i.at[w]], gbib, sem),
        )
        for cp in cps:
            cp.start()
        for cp in cps:
            cp.wait()

        def grp(g, _, w=w):
            sl = pl.ds(w * WAVE + g * 16, 16)
            lsl = pl.ds(g * 16, 16)
            u16 = iv_u[sl]
            i16 = iv_i[sl]
            rr16 = g * 16 + lanes
            offu = (u16 & 1) * D
            offi = (i16 & 1) * D

            def col(c, acc):
                vu = plsc.load_gather(gbu, [rr16, offu + c])
                vi = plsc.load_gather(gbi, [rr16, offi + c])
                return acc + vu * vi

            acc0 = (plsc.load_gather(gbub, [rr16, u16 & 127])
                    + plsc.load_gather(gbib, [rr16, i16 & 127]))
            x = lax.fori_loop(0, D, col, acc0)
            out_v[sl] = 1.0 / (1.0 + jnp.exp(-x))
            return 0

        lax.fori_loop(0, WAVE // 16, grp, 0)

    pltpu.sync_copy(out_v, out_hbm.at[pl.ds(base, BPW)])


@functools.partial(jax.jit, static_argnames=())
def _run(u_idx, i_idx, ut2, ubp, it2, ibp):
    mesh = plsc.VectorSubcoreMesh(core_axis_name="c", subcore_axis_name="s",
                                  num_cores=NC, num_subcores=NS)
    f = pl.kernel(
        _body,
        out_type=jax.ShapeDtypeStruct((B,), jnp.float32),
        mesh=mesh,
        compiler_params=pltpu.CompilerParams(needs_layout_passes=False,
                                             use_tc_tiling_on_sc=True),
        scratch_types=[
            pltpu.VMEM((BPW,), jnp.int32),
            pltpu.VMEM((BPW,), jnp.int32),
            pltpu.VMEM((NWAVE, WAVE), jnp.int32),
            pltpu.VMEM((NWAVE, WAVE), jnp.int32),
            pltpu.VMEM((NWAVE, WAVE), jnp.int32),
            pltpu.VMEM((NWAVE, WAVE), jnp.int32),
            pltpu.VMEM((WAVE, 128), jnp.float32),
            pltpu.VMEM((WAVE, 128), jnp.float32),
            pltpu.VMEM((WAVE, 128), jnp.float32),
            pltpu.VMEM((WAVE, 128), jnp.float32),
            pltpu.VMEM((BPW,), jnp.float32),
            pltpu.SemaphoreType.DMA,
        ],
    )
    return f(u_idx, i_idx, ut2, ubp, it2, ibp)


NCOL = NBROW
CPW = (NCOL + NW - 1) // NW


def _conv_body(src_hbm, out_hbm,
               ib0, ib1, ob0, ob1, si0, si1, so0, so1):
    wid = lax.axis_index("s") * NC + lax.axis_index("c")
    t0 = wid * CPW
    nch = jnp.minimum(CPW, NCOL - t0)
    lanes = lax.iota(jnp.int32, 16)
    ibufs = (ib0, ib1)
    obufs = (ob0, ob1)
    sis = (si0, si1)
    sos = (so0, so1)

    def in_copy(t_local, b):
        tg = pl.multiple_of((t0 + t_local) * 128, 128)
        return pltpu.make_async_copy(src_hbm.at[:, pl.ds(tg, 128)],
                                     ibufs[b], sis[b])

    def out_copy(t_local, b):
        rg = pl.multiple_of((t0 + t_local) * 64, 64)
        return pltpu.make_async_copy(obufs[b], out_hbm.at[pl.ds(rg, 64), :],
                                     sos[b])

    def step(m, _):
        for b in range(2):
            k = 2 * m + b

            @pl.when(k < nch)
            def _():
                in_copy(k, b).start()

            c = k - 1
            bc = 1 - b

            @pl.when((c >= 0) & (c < nch))
            def _():
                in_copy(c, bc).wait()

                @pl.when(c >= 2)
                def _():
                    out_copy(c - 2, bc).wait()

                def row(r, _):
                    le = jnp.full((16,), 2 * r, jnp.int32)
                    lo = jnp.full((16,), 2 * r + 1, jnp.int32)
                    for k4 in range(4):
                        fv = k4 * 16 + lanes
                        obufs[bc][r, pl.ds(k4 * 16, 16)] = (
                            plsc.load_gather(ibufs[bc], [fv, le]))
                        obufs[bc][r, pl.ds(64 + k4 * 16, 16)] = (
                            plsc.load_gather(ibufs[bc], [fv, lo]))
                    return 0

                lax.fori_loop(0, 64, row, 0)
                out_copy(c, bc).start()

        return 0

    lax.fori_loop(0, (CPW + 2) // 2 + 1, step, 0)

    for b in range(2):
        cb = ((nch - 1 - b) // 2) * 2 + b

        @pl.when(cb >= 0)
        def _(cb=cb, b=b):
            out_copy(cb, b).wait()


@functools.partial(jax.jit, static_argnames=())
def _convert(src):
    mesh = plsc.VectorSubcoreMesh(core_axis_name="c", subcore_axis_name="s",
                                  num_cores=NC, num_subcores=NS)
    f = pl.kernel(
        _conv_body,
        out_type=jax.ShapeDtypeStruct((NCOL * 64, 128), jnp.float32),
        mesh=mesh,
        compiler_params=pltpu.CompilerParams(needs_layout_passes=False,
                                             use_tc_tiling_on_sc=True),
        scratch_types=[
            pltpu.VMEM((D, 128), jnp.float32),
            pltpu.VMEM((D, 128), jnp.float32),
            pltpu.VMEM((D, 128), jnp.float32),
            pltpu.VMEM((D, 128), jnp.float32),
            pltpu.SemaphoreType.DMA,
            pltpu.SemaphoreType.DMA,
            pltpu.SemaphoreType.DMA,
            pltpu.SemaphoreType.DMA,
        ],
    )
    return f(src)


def kernel(inputs, user_embedding, user_bias, item_embedding, item_bias):
    u_idx = inputs[:, 0]
    i_idx = inputs[:, 1]
    ut2 = _convert(user_embedding.T)
    it2 = _convert(item_embedding.T)
    ubp = jnp.pad(user_bias[:, 0], (0, NBROW * 128 - user_bias.shape[0])
                  ).reshape(NBROW, 128)
    ibp = jnp.pad(item_bias[:, 0], (0, NBROW * 128 - item_bias.shape[0])
                  ).reshape(NBROW, 128)
    out = _run(u_idx, i_idx, ut2, ubp, it2, ibp)
    return out[:, None]


# speedup vs baseline: 2.6569x; 2.6569x over previous
import functools

import jax
import jax.numpy as jnp
from jax import lax
from jax.experimental import pallas as pl
from jax.experimental.pallas import tpu as pltpu
from jax.experimental.pallas import tpu_sc as plsc

B = 16384
D = 64
NC = 2
NS = 16
NW = NC * NS
BPW = B // NW
WAVE = 128
NWAVE = BPW // WAVE
NBROW = 7813


def _body(u_idx_hbm, i_idx_hbm, ut2_hbm, ubp_hbm, it2_hbm, ibp_hbm,
          out_hbm,
          iv_u, iv_i, r2u, r2i, r3u, r3i,
          gbu, gbi, gbub, gbib, out_v, sem):
    wid = lax.axis_index("s") * NC + lax.axis_index("c")
    base = wid * BPW

    pltpu.sync_copy(u_idx_hbm.at[pl.ds(base, BPW)], iv_u)
    pltpu.sync_copy(i_idx_hbm.at[pl.ds(base, BPW)], iv_i)

    for k in range(BPW // 16):
        w, off = k // (WAVE // 16), (k % (WAVE // 16)) * 16
        sl = pl.ds(k * 16, 16)
        dsl = pl.ds(off, 16)
        u16 = iv_u[sl]
        i16 = iv_i[sl]
        r2u[w, dsl] = lax.shift_right_logical(u16, 1)
        r2i[w, dsl] = lax.shift_right_logical(i16, 1)
        r3u[w, dsl] = lax.shift_right_logical(u16, 7)
        r3i[w, dsl] = lax.shift_right_logical(i16, 7)

    lanes = lax.iota(jnp.int32, 16)

    for w in range(NWAVE):
        cps = (
            pltpu.make_async_copy(ut2_hbm.at[r2u.at[w]], gbu, sem),
            pltpu.make_async_copy(it2_hbm.at[r2i.at[w]], gbi, sem),
            pltpu.make_async_copy(ubp_hbm.at[r3u.at[w]], gbub, sem),
            pltpu.make_async_copy(ibp_hbm.at[r3i.at[w]], gbib, sem),
        )
        for cp in cps:
            cp.start()
        for cp in cps:
            cp.wait()

        def grp(g, _, w=w):
            sl = pl.ds(w * WAVE + g * 16, 16)
            lsl = pl.ds(g * 16, 16)
            u16 = iv_u[sl]
            i16 = iv_i[sl]
            rr16 = g * 16 + lanes
            offu = (u16 & 1) * D
            offi = (i16 & 1) * D

            def col(c, acc):
                vu = plsc.load_gather(gbu, [rr16, offu + c])
                vi = plsc.load_gather(gbi, [rr16, offi + c])
                return acc + vu * vi

            acc0 = (plsc.load_gather(gbub, [rr16, u16 & 127])
                    + plsc.load_gather(gbib, [rr16, i16 & 127]))
            x = lax.fori_loop(0, D, col, acc0)
            out_v[sl] = 1.0 / (1.0 + jnp.exp(-x))
            return 0

        lax.fori_loop(0, WAVE // 16, grp, 0)

    pltpu.sync_copy(out_v, out_hbm.at[pl.ds(base, BPW)])


@functools.partial(jax.jit, static_argnames=())
def _run(u_idx, i_idx, ut2, ubp, it2, ibp):
    mesh = plsc.VectorSubcoreMesh(core_axis_name="c", subcore_axis_name="s",
                                  num_cores=NC, num_subcores=NS)
    f = pl.kernel(
        _body,
        out_type=jax.ShapeDtypeStruct((B,), jnp.float32),
        mesh=mesh,
        compiler_params=pltpu.CompilerParams(needs_layout_passes=False,
                                             use_tc_tiling_on_sc=True),
        scratch_types=[
            pltpu.VMEM((BPW,), jnp.int32),
            pltpu.VMEM((BPW,), jnp.int32),
            pltpu.VMEM((NWAVE, WAVE), jnp.int32),
            pltpu.VMEM((NWAVE, WAVE), jnp.int32),
            pltpu.VMEM((NWAVE, WAVE), jnp.int32),
            pltpu.VMEM((NWAVE, WAVE), jnp.int32),
            pltpu.VMEM((WAVE, 128), jnp.float32),
            pltpu.VMEM((WAVE, 128), jnp.float32),
            pltpu.VMEM((WAVE, 128), jnp.float32),
            pltpu.VMEM((WAVE, 128), jnp.float32),
            pltpu.VMEM((BPW,), jnp.float32),
            pltpu.SemaphoreType.DMA,
        ],
    )
    return f(u_idx, i_idx, ut2, ubp, it2, ibp)


def kernel(inputs, user_embedding, user_bias, item_embedding, item_bias):
    u_idx = inputs[:, 0]
    i_idx = inputs[:, 1]
    ut2 = user_embedding.reshape(-1, 128)
    it2 = item_embedding.reshape(-1, 128)
    ubp = jnp.pad(user_bias[:, 0], (0, NBROW * 128 - user_bias.shape[0])
                  ).reshape(NBROW, 128)
    ibp = jnp.pad(item_bias[:, 0], (0, NBROW * 128 - item_bias.shape[0])
                  ).reshape(NBROW, 128)
    out = _run(u_idx, i_idx, ut2, ubp, it2, ibp)
    return out[:, None]
